# initial kernel scaffold (unmeasured)
import math

import jax
import jax.numpy as jnp
from jax import lax
from jax.experimental import pallas as pl
from jax.experimental.pallas import tpu as pltpu

N_DEV = 8
B = 2
S_PER = 256
S = N_DEV * S_PER
D = 768
H_PER = 4
DH = 64
N_HOP = N_DEV - 1


def kernel(x, Wq, Wk, Wv, Wo):
    bf16 = jnp.bfloat16
    f32 = jnp.float32

    x_b = x.astype(bf16)
    wq = Wq.reshape(D, H_PER, DH).transpose(1, 0, 2).astype(bf16)
    wk = Wk.reshape(D, H_PER, DH).transpose(1, 0, 2).astype(bf16)
    wv = Wv.reshape(D, H_PER, DH).transpose(1, 0, 2).astype(bf16)
    wo = Wo.astype(bf16)

    def body(x_ref, wq_ref, wk_ref, wv_ref, wo_ref, out_ref,
             xg_ref, acc_ref, rs_ref, ag_send, ag_recv, rs_send, rs_recv):
        my = lax.axis_index("i")
        left = lax.rem(my + N_DEV - 1, N_DEV)
        right = lax.rem(my + 1, N_DEV)

        barrier = pltpu.get_barrier_semaphore()
        for nbr in (left, right):
            pl.semaphore_signal(barrier, inc=1, device_id=(nbr,),
                                device_id_type=pl.DeviceIdType.MESH)
        pl.semaphore_wait(barrier, 2)

        xg_ref[:, pl.ds(my * S_PER, S_PER), :] = x_ref[...]
        for h in range(N_HOP):
            o = lax.rem(my - h + N_DEV, N_DEV)
            rdma = pltpu.make_async_remote_copy(
                src_ref=xg_ref.at[:, pl.ds(o * S_PER, S_PER), :],
                dst_ref=xg_ref.at[:, pl.ds(o * S_PER, S_PER), :],
                send_sem=ag_send.at[h],
                recv_sem=ag_recv.at[h],
                device_id=(right,),
                device_id_type=pl.DeviceIdType.MESH,
            )
            rdma.start()
            rdma.wait()

        pos = lax.broadcasted_iota(f32, (S, DH), 0)
        lane = lax.broadcasted_iota(jnp.int32, (S, DH), 1)
        even = lane - lax.rem(lane, 2)
        inv = jnp.exp(even.astype(f32) * (-math.log(10000.0) / DH))
        theta = pos * inv
        cos_t = jnp.cos(theta)
        sin_t = jnp.sin(theta)
        r_row = lax.broadcasted_iota(jnp.int32, (DH, DH), 0)
        r_col = lax.broadcasted_iota(jnp.int32, (DH, DH), 1)
        rot_m = jnp.where(
            (lax.rem(r_col, 2) == 0) & (r_row == r_col + 1), -1.0,
            jnp.where((lax.rem(r_col, 2) == 1) & (r_row == r_col - 1), 1.0, 0.0),
        ).astype(f32)

        def rope(t):
            return (t * cos_t
                    + jnp.dot(t, rot_m, preferred_element_type=f32) * sin_t)

        for b in range(B):
            xb = xg_ref[b]
            ctxs = []
            for h in range(H_PER):
                q = jnp.dot(xb, wq_ref[h], preferred_element_type=f32)
                k = jnp.dot(xb, wk_ref[h], preferred_element_type=f32)
                v = jnp.dot(xb, wv_ref[h], preferred_element_type=f32)
                q = rope(q).astype(bf16)
                k = rope(k).astype(bf16)
                s = lax.dot_general(
                    q, k, (((1,), (1,)), ((), ())),
                    preferred_element_type=f32,
                ) * 0.125
                m = jnp.max(s, axis=1, keepdims=True)
                e = jnp.exp(s - m)
                w = (e / jnp.sum(e, axis=1, keepdims=True)).astype(bf16)
                ctxs.append(
                    jnp.dot(w, v.astype(bf16), preferred_element_type=f32)
                    .astype(bf16))
            ctx = jnp.concatenate(ctxs, axis=1)
            acc_ref[b] = jnp.dot(ctx, wo_ref[...], preferred_element_type=f32)

        c0 = lax.rem(my + N_DEV - 1, N_DEV)
        rs_ref[N_HOP] = acc_ref[:, pl.ds(c0 * S_PER, S_PER), :].astype(bf16)
        for s in range(N_HOP):
            src_slot = N_HOP if s == 0 else s - 1
            rdma = pltpu.make_async_remote_copy(
                src_ref=rs_ref.at[src_slot],
                dst_ref=rs_ref.at[s],
                send_sem=rs_send.at[s],
                recv_sem=rs_recv.at[s],
                device_id=(right,),
                device_id_type=pl.DeviceIdType.MESH,
            )
            rdma.start()
            rdma.wait()
            c = lax.rem(my - s - 2 + 2 * N_DEV, N_DEV)
            part = acc_ref[:, pl.ds(c * S_PER, S_PER), :]
            got = rs_ref[s].astype(f32)
            if s < N_HOP - 1:
                rs_ref[s] = (got + part).astype(bf16)
            else:
                out_ref[...] = got + part

    return pl.pallas_call(
        body,
        out_shape=jax.ShapeDtypeStruct((B, S_PER, D), jnp.float32),
        in_specs=[pl.BlockSpec(memory_space=pltpu.VMEM)] * 5,
        out_specs=pl.BlockSpec(memory_space=pltpu.VMEM),
        scratch_shapes=[
            pltpu.VMEM((B, S, D), bf16),
            pltpu.VMEM((B, S, D), f32),
            pltpu.VMEM((N_DEV, B, S_PER, D), bf16),
            pltpu.SemaphoreType.DMA((N_HOP,)),
            pltpu.SemaphoreType.DMA((N_HOP,)),
            pltpu.SemaphoreType.DMA((N_HOP,)),
            pltpu.SemaphoreType.DMA((N_HOP,)),
        ],
        compiler_params=pltpu.CompilerParams(collective_id=0),
    )(x_b, wq, wk, wv, wo)


# baseline (device time: 255100 ns/iter reference)
import math

import jax
import jax.numpy as jnp
from jax import lax
from jax.experimental import pallas as pl
from jax.experimental.pallas import tpu as pltpu

N_DEV = 8
B = 2
S_PER = 256
S = N_DEV * S_PER
D = 768
H_PER = 4
DH = 64
N_HOP = N_DEV - 1


def kernel(x, Wq, Wk, Wv, Wo):
    bf16 = jnp.bfloat16
    f32 = jnp.float32

    x_b = x.astype(bf16)
    wq = Wq.reshape(D, H_PER, DH).transpose(1, 0, 2).astype(bf16)
    wk = Wk.reshape(D, H_PER, DH).transpose(1, 0, 2).astype(bf16)
    wv = Wv.reshape(D, H_PER, DH).transpose(1, 0, 2).astype(bf16)
    wo = Wo.astype(bf16)

    def body(x_ref, wq_ref, wk_ref, wv_ref, wo_ref, out_ref,
             xg_ref, acc_ref, rs_ref, ag_send, ag_recv, rs_send, rs_recv):
        my = lax.axis_index("i")
        left = lax.rem(my + N_DEV - 1, N_DEV)
        right = lax.rem(my + 1, N_DEV)

        barrier = pltpu.get_barrier_semaphore()
        for nbr in (left, right):
            pl.semaphore_signal(barrier, inc=1, device_id=(nbr,),
                                device_id_type=pl.DeviceIdType.MESH)
        pl.semaphore_wait(barrier, 2)

        xg_ref[:, pl.ds(my * S_PER, S_PER), :] = x_ref[...]
        for h in range(N_HOP):
            o = lax.rem(my - h + N_DEV, N_DEV)
            rdma = pltpu.make_async_remote_copy(
                src_ref=xg_ref.at[:, pl.ds(o * S_PER, S_PER), :],
                dst_ref=xg_ref.at[:, pl.ds(o * S_PER, S_PER), :],
                send_sem=ag_send.at[h],
                recv_sem=ag_recv.at[h],
                device_id=(right,),
                device_id_type=pl.DeviceIdType.MESH,
            )
            rdma.start()
            rdma.wait()

        pos = lax.broadcasted_iota(jnp.int32, (S, DH), 0).astype(f32)
        lane = lax.broadcasted_iota(jnp.int32, (S, DH), 1)
        even = lane - lax.rem(lane, 2)
        inv = jnp.exp(even.astype(f32) * (-math.log(10000.0) / DH))
        theta = pos * inv
        cos_t = jnp.cos(theta)
        sin_t = jnp.sin(theta)
        r_row = lax.broadcasted_iota(jnp.int32, (DH, DH), 0)
        r_col = lax.broadcasted_iota(jnp.int32, (DH, DH), 1)
        rot_m = jnp.where(
            (lax.rem(r_col, 2) == 0) & (r_row == r_col + 1), -1.0,
            jnp.where((lax.rem(r_col, 2) == 1) & (r_row == r_col - 1), 1.0, 0.0),
        ).astype(f32)

        def rope(t):
            return (t * cos_t
                    + jnp.dot(t, rot_m, preferred_element_type=f32) * sin_t)

        for b in range(B):
            xb = xg_ref[b]
            ctxs = []
            for h in range(H_PER):
                q = jnp.dot(xb, wq_ref[h], preferred_element_type=f32)
                k = jnp.dot(xb, wk_ref[h], preferred_element_type=f32)
                v = jnp.dot(xb, wv_ref[h], preferred_element_type=f32)
                q = rope(q).astype(bf16)
                k = rope(k).astype(bf16)
                s = lax.dot_general(
                    q, k, (((1,), (1,)), ((), ())),
                    preferred_element_type=f32,
                ) * 0.125
                m = jnp.max(s, axis=1, keepdims=True)
                e = jnp.exp(s - m)
                w = (e / jnp.sum(e, axis=1, keepdims=True)).astype(bf16)
                ctxs.append(
                    jnp.dot(w, v.astype(bf16), preferred_element_type=f32)
                    .astype(bf16))
            ctx = jnp.concatenate(ctxs, axis=1)
            acc_ref[b] = jnp.dot(ctx, wo_ref[...], preferred_element_type=f32)

        c0 = lax.rem(my + N_DEV - 1, N_DEV)
        rs_ref[N_HOP] = acc_ref[:, pl.ds(c0 * S_PER, S_PER), :].astype(bf16)
        for s in range(N_HOP):
            src_slot = N_HOP if s == 0 else s - 1
            rdma = pltpu.make_async_remote_copy(
                src_ref=rs_ref.at[src_slot],
                dst_ref=rs_ref.at[s],
                send_sem=rs_send.at[s],
                recv_sem=rs_recv.at[s],
                device_id=(right,),
                device_id_type=pl.DeviceIdType.MESH,
            )
            rdma.start()
            rdma.wait()
            c = lax.rem(my - s - 2 + 2 * N_DEV, N_DEV)
            part = acc_ref[:, pl.ds(c * S_PER, S_PER), :]
            got = rs_ref[s].astype(f32)
            if s < N_HOP - 1:
                rs_ref[s] = (got + part).astype(bf16)
            else:
                out_ref[...] = got + part

    return pl.pallas_call(
        body,
        out_shape=jax.ShapeDtypeStruct((B, S_PER, D), jnp.float32),
        in_specs=[pl.BlockSpec(memory_space=pltpu.VMEM)] * 5,
        out_specs=pl.BlockSpec(memory_space=pltpu.VMEM),
        scratch_shapes=[
            pltpu.VMEM((B, S, D), bf16),
            pltpu.VMEM((B, S, D), f32),
            pltpu.VMEM((N_DEV, B, S_PER, D), bf16),
            pltpu.SemaphoreType.DMA((N_HOP,)),
            pltpu.SemaphoreType.DMA((N_HOP,)),
            pltpu.SemaphoreType.DMA((N_HOP,)),
            pltpu.SemaphoreType.DMA((N_HOP,)),
        ],
        compiler_params=pltpu.CompilerParams(
            collective_id=0, vmem_limit_bytes=110 * 1024 * 1024),
    )(x_b, wq, wk, wv, wo)


# device time: 177573 ns/iter; 1.4366x vs baseline; 1.4366x over previous
import math

import jax
import jax.numpy as jnp
from jax import lax
from jax.experimental import pallas as pl
from jax.experimental.pallas import tpu as pltpu

N_DEV = 8
B = 2
S_PER = 256
S = N_DEV * S_PER
D = 768
H_PER = 4
DH = 64
N_HOP = N_DEV - 1


def kernel(x, Wq, Wk, Wv, Wo):
    bf16 = jnp.bfloat16
    f32 = jnp.float32

    x_b = x.astype(bf16)
    wq = Wq.reshape(D, H_PER, DH).transpose(1, 0, 2).astype(bf16)
    wk = Wk.reshape(D, H_PER, DH).transpose(1, 0, 2).astype(bf16)
    wv = Wv.reshape(D, H_PER, DH).transpose(1, 0, 2).astype(bf16)
    wo = Wo.astype(bf16)

    def body(x_ref, wq_ref, wk_ref, wv_ref, wo_ref, out_ref,
             xg_ref, qb_ref, kb_ref, vb_ref, rs_ref,
             ag_send, ag_recv, rs_send, rs_recv):
        my = lax.axis_index("i")
        left = lax.rem(my + N_DEV - 1, N_DEV)
        right = lax.rem(my + 1, N_DEV)

        barrier = pltpu.get_barrier_semaphore()
        for nbr in (left, right):
            pl.semaphore_signal(barrier, inc=1, device_id=(nbr,),
                                device_id_type=pl.DeviceIdType.MESH)
        pl.semaphore_wait(barrier, 2)

        def project(o):
            base = (o * S_PER).astype(f32)
            pos = lax.broadcasted_iota(jnp.int32, (S_PER, DH), 0).astype(f32)
            pos = pos + base
            lanes = lax.broadcasted_iota(jnp.int32, (S_PER, DH), 1)
            even = lanes - lax.rem(lanes, 2)
            theta = pos * jnp.exp(even.astype(f32) * (-math.log(10000.0) / DH))
            cos_t = jnp.cos(theta)
            sin_t = jnp.sin(theta)
            r_row = lax.broadcasted_iota(jnp.int32, (DH, DH), 0)
            r_col = lax.broadcasted_iota(jnp.int32, (DH, DH), 1)
            rot_m = jnp.where(
                (lax.rem(r_col, 2) == 0) & (r_row == r_col + 1), -1.0,
                jnp.where(
                    (lax.rem(r_col, 2) == 1) & (r_row == r_col - 1), 1.0, 0.0),
            ).astype(f32)

            def rope(t):
                return (t * cos_t
                        + jnp.dot(t, rot_m, preferred_element_type=f32) * sin_t)

            for b in range(B):
                xo = xg_ref[b, pl.ds(o * S_PER, S_PER), :]
                for h in range(H_PER):
                    q = jnp.dot(xo, wq_ref[h], preferred_element_type=f32)
                    k = jnp.dot(xo, wk_ref[h], preferred_element_type=f32)
                    v = jnp.dot(xo, wv_ref[h], preferred_element_type=f32)
                    qb_ref[b, h, pl.ds(o * S_PER, S_PER), :] = rope(q).astype(bf16)
                    kb_ref[b, h, pl.ds(o * S_PER, S_PER), :] = rope(k).astype(bf16)
                    vb_ref[b, h, pl.ds(o * S_PER, S_PER), :] = v.astype(bf16)

        xg_ref[:, pl.ds(my * S_PER, S_PER), :] = x_ref[...]
        project(my)

        def ag_hop(h, carry):
            o = lax.rem(my - h + 2 * N_DEV, N_DEV)
            rdma = pltpu.make_async_remote_copy(
                src_ref=xg_ref.at[:, pl.ds(o * S_PER, S_PER), :],
                dst_ref=xg_ref.at[:, pl.ds(o * S_PER, S_PER), :],
                send_sem=ag_send.at[h],
                recv_sem=ag_recv.at[h],
                device_id=(right,),
                device_id_type=pl.DeviceIdType.MESH,
            )
            rdma.start()
            @pl.when(h > 0)
            def _():
                project(o)
            rdma.wait()
            return carry

        lax.fori_loop(0, N_HOP, ag_hop, 0)
        project(lax.rem(my - N_HOP + 2 * N_DEV, N_DEV))

        def attn_chunk(c, b):
            ctxs = []
            for h in range(H_PER):
                qc = qb_ref[b, h, pl.ds(c * S_PER, S_PER), :]
                kh = kb_ref[b, h]
                s_ = lax.dot_general(
                    qc, kh, (((1,), (1,)), ((), ())),
                    preferred_element_type=f32,
                ) * 0.125
                m = jnp.max(s_, axis=1, keepdims=True)
                e = jnp.exp(s_ - m)
                w = (e / jnp.sum(e, axis=1, keepdims=True)).astype(bf16)
                ctxs.append(
                    jnp.dot(w, vb_ref[b, h], preferred_element_type=f32)
                    .astype(bf16))
            ctx = jnp.concatenate(ctxs, axis=1)
            return jnp.dot(ctx, wo_ref[...], preferred_element_type=f32)

        rs_ref[N_HOP] = jnp.zeros((B, S_PER, D), bf16)

        def rs_hop(s, carry):
            c = lax.rem(my - 1 - s + 2 * N_DEV, N_DEV)
            sm1 = jnp.maximum(s - 1, 0)
            src = jnp.where(s == 0, N_HOP, sm1)
            p0 = attn_chunk(c, 0)
            p1 = attn_chunk(c, 1)

            @pl.when(s > 0)
            def _():
                prev = pltpu.make_async_remote_copy(
                    src_ref=rs_ref.at[sm1],
                    dst_ref=rs_ref.at[sm1],
                    send_sem=rs_send.at[sm1],
                    recv_sem=rs_recv.at[sm1],
                    device_id=(right,),
                    device_id_type=pl.DeviceIdType.MESH,
                )
                prev.wait_recv()

            rs_ref[src, 0] = (rs_ref[src, 0].astype(f32) + p0).astype(bf16)
            rs_ref[src, 1] = (rs_ref[src, 1].astype(f32) + p1).astype(bf16)
            rdma = pltpu.make_async_remote_copy(
                src_ref=rs_ref.at[src],
                dst_ref=rs_ref.at[s],
                send_sem=rs_send.at[s],
                recv_sem=rs_recv.at[s],
                device_id=(right,),
                device_id_type=pl.DeviceIdType.MESH,
            )
            rdma.start()
            return carry

        lax.fori_loop(0, N_HOP, rs_hop, 0)

        p0 = attn_chunk(my, 0)
        p1 = attn_chunk(my, 1)
        last = pltpu.make_async_remote_copy(
            src_ref=rs_ref.at[N_HOP],
            dst_ref=rs_ref.at[N_HOP - 1],
            send_sem=rs_send.at[N_HOP - 1],
            recv_sem=rs_recv.at[N_HOP - 1],
            device_id=(right,),
            device_id_type=pl.DeviceIdType.MESH,
        )
        last.wait_recv()
        out_ref[0] = rs_ref[N_HOP - 1, 0].astype(f32) + p0
        out_ref[1] = rs_ref[N_HOP - 1, 1].astype(f32) + p1

        for s in range(N_HOP):
            d = pltpu.make_async_remote_copy(
                src_ref=rs_ref.at[s],
                dst_ref=rs_ref.at[s],
                send_sem=rs_send.at[s],
                recv_sem=rs_recv.at[s],
                device_id=(right,),
                device_id_type=pl.DeviceIdType.MESH,
            )
            d.wait_send()

    return pl.pallas_call(
        body,
        out_shape=jax.ShapeDtypeStruct((B, S_PER, D), jnp.float32),
        in_specs=[pl.BlockSpec(memory_space=pltpu.VMEM)] * 5,
        out_specs=pl.BlockSpec(memory_space=pltpu.VMEM),
        scratch_shapes=[
            pltpu.VMEM((B, S, D), bf16),
            pltpu.VMEM((B, H_PER, S, DH), bf16),
            pltpu.VMEM((B, H_PER, S, DH), bf16),
            pltpu.VMEM((B, H_PER, S, DH), bf16),
            pltpu.VMEM((N_DEV, B, S_PER, D), bf16),
            pltpu.SemaphoreType.DMA((N_HOP,)),
            pltpu.SemaphoreType.DMA((N_HOP,)),
            pltpu.SemaphoreType.DMA((N_HOP,)),
            pltpu.SemaphoreType.DMA((N_HOP,)),
        ],
        compiler_params=pltpu.CompilerParams(
            collective_id=0, vmem_limit_bytes=110 * 1024 * 1024),
    )(x_b, wq, wk, wv, wo)


# device time: 147712 ns/iter; 1.7270x vs baseline; 1.2022x over previous
import math

import jax
import jax.numpy as jnp
from jax import lax
from jax.experimental import pallas as pl
from jax.experimental.pallas import tpu as pltpu

N_DEV = 8
B = 2
S_PER = 256
S = N_DEV * S_PER
D = 768
H_PER = 4
DH = 64
N_HOP = N_DEV - 1


def kernel(x, Wq, Wk, Wv, Wo):
    bf16 = jnp.bfloat16
    f32 = jnp.float32

    x_b = x.astype(bf16)
    wq = Wq.reshape(D, H_PER, DH).transpose(1, 0, 2).astype(bf16)
    wk = Wk.reshape(D, H_PER, DH).transpose(1, 0, 2).astype(bf16)
    wv = Wv.reshape(D, H_PER, DH).transpose(1, 0, 2).astype(bf16)
    wo = Wo.astype(bf16)

    def body(x_ref, wq_ref, wk_ref, wv_ref, wo_ref, out_ref,
             xg_ref, qb_ref, kb_ref, vb_ref, rsr_ref, rsl_ref,
             agr_send, agr_recv, agl_send, agl_recv,
             rsr_send, rsr_recv, rsl_send, rsl_recv):
        my = lax.axis_index("i")
        left = lax.rem(my + N_DEV - 1, N_DEV)
        right = lax.rem(my + 1, N_DEV)

        barrier = pltpu.get_barrier_semaphore()
        for nbr in (left, right):
            pl.semaphore_signal(barrier, inc=1, device_id=(nbr,),
                                device_id_type=pl.DeviceIdType.MESH)
        pl.semaphore_wait(barrier, 2)

        def project(o, b):
            base = (o * S_PER).astype(f32)
            pos = lax.broadcasted_iota(jnp.int32, (S_PER, DH), 0).astype(f32)
            pos = pos + base
            lanes = lax.broadcasted_iota(jnp.int32, (S_PER, DH), 1)
            even = lanes - lax.rem(lanes, 2)
            theta = pos * jnp.exp(even.astype(f32) * (-math.log(10000.0) / DH))
            cos_t = jnp.cos(theta)
            sin_t = jnp.sin(theta)
            r_row = lax.broadcasted_iota(jnp.int32, (DH, DH), 0)
            r_col = lax.broadcasted_iota(jnp.int32, (DH, DH), 1)
            rot_m = jnp.where(
                (lax.rem(r_col, 2) == 0) & (r_row == r_col + 1), -1.0,
                jnp.where(
                    (lax.rem(r_col, 2) == 1) & (r_row == r_col - 1), 1.0, 0.0),
            ).astype(f32)

            def rope(t):
                return (t * cos_t
                        + jnp.dot(t, rot_m, preferred_element_type=f32) * sin_t)

            xo = xg_ref[b, pl.ds(o * S_PER, S_PER), :]
            for h in range(H_PER):
                q = jnp.dot(xo, wq_ref[h], preferred_element_type=f32)
                k = jnp.dot(xo, wk_ref[h], preferred_element_type=f32)
                v = jnp.dot(xo, wv_ref[h], preferred_element_type=f32)
                qb_ref[b, h, pl.ds(o * S_PER, S_PER), :] = rope(q).astype(bf16)
                kb_ref[b, h, pl.ds(o * S_PER, S_PER), :] = rope(k).astype(bf16)
                vb_ref[b, h, pl.ds(o * S_PER, S_PER), :] = v.astype(bf16)

        xg_ref[:, pl.ds(my * S_PER, S_PER), :] = x_ref[...]
        project(my, 0)
        project(my, 1)

        def ag_hop(h, carry):
            o_r = lax.rem(my - h + 2 * N_DEV, N_DEV)
            o_l = lax.rem(my + h, N_DEV)
            rdma_r = pltpu.make_async_remote_copy(
                src_ref=xg_ref.at[0, pl.ds(o_r * S_PER, S_PER), :],
                dst_ref=xg_ref.at[0, pl.ds(o_r * S_PER, S_PER), :],
                send_sem=agr_send.at[h],
                recv_sem=agr_recv.at[h],
                device_id=(right,),
                device_id_type=pl.DeviceIdType.MESH,
            )
            rdma_l = pltpu.make_async_remote_copy(
                src_ref=xg_ref.at[1, pl.ds(o_l * S_PER, S_PER), :],
                dst_ref=xg_ref.at[1, pl.ds(o_l * S_PER, S_PER), :],
                send_sem=agl_send.at[h],
                recv_sem=agl_recv.at[h],
                device_id=(left,),
                device_id_type=pl.DeviceIdType.MESH,
            )
            rdma_r.start()
            rdma_l.start()
            @pl.when(h > 0)
            def _():
                project(o_r, 0)
                project(o_l, 1)
            rdma_r.wait()
            rdma_l.wait()
            return carry

        lax.fori_loop(0, N_HOP, ag_hop, 0)
        project(lax.rem(my + 1, N_DEV), 0)
        project(lax.rem(my + N_DEV - 1, N_DEV), 1)

        def attn_chunk(c, b):
            ctxs = []
            for h in range(H_PER):
                qc = qb_ref[b, h, pl.ds(c * S_PER, S_PER), :]
                kh = kb_ref[b, h]
                s_ = lax.dot_general(
                    qc, kh, (((1,), (1,)), ((), ())),
                    preferred_element_type=f32,
                ) * 0.125
                m = jnp.max(s_, axis=1, keepdims=True)
                e = jnp.exp(s_ - m)
                w = (e / jnp.sum(e, axis=1, keepdims=True)).astype(bf16)
                ctxs.append(
                    jnp.dot(w, vb_ref[b, h], preferred_element_type=f32)
                    .astype(bf16))
            ctx = jnp.concatenate(ctxs, axis=1)
            return jnp.dot(ctx, wo_ref[...], preferred_element_type=f32)

        rsr_ref[N_HOP] = jnp.zeros((S_PER, D), bf16)
        rsl_ref[N_HOP] = jnp.zeros((S_PER, D), bf16)

        def rs_hop(s, carry):
            c_r = lax.rem(my - 1 - s + 2 * N_DEV, N_DEV)
            c_l = lax.rem(my + 1 + s, N_DEV)
            sm1 = jnp.maximum(s - 1, 0)
            src = jnp.where(s == 0, N_HOP, sm1)
            p_r = attn_chunk(c_r, 0)
            p_l = attn_chunk(c_l, 1)

            @pl.when(s > 0)
            def _():
                prev_r = pltpu.make_async_remote_copy(
                    src_ref=rsr_ref.at[sm1], dst_ref=rsr_ref.at[sm1],
                    send_sem=rsr_send.at[sm1], recv_sem=rsr_recv.at[sm1],
                    device_id=(right,), device_id_type=pl.DeviceIdType.MESH,
                )
                prev_l = pltpu.make_async_remote_copy(
                    src_ref=rsl_ref.at[sm1], dst_ref=rsl_ref.at[sm1],
                    send_sem=rsl_send.at[sm1], recv_sem=rsl_recv.at[sm1],
                    device_id=(left,), device_id_type=pl.DeviceIdType.MESH,
                )
                prev_r.wait_recv()
                prev_l.wait_recv()

            rsr_ref[src] = (rsr_ref[src].astype(f32) + p_r).astype(bf16)
            rsl_ref[src] = (rsl_ref[src].astype(f32) + p_l).astype(bf16)
            rdma_r = pltpu.make_async_remote_copy(
                src_ref=rsr_ref.at[src], dst_ref=rsr_ref.at[s],
                send_sem=rsr_send.at[s], recv_sem=rsr_recv.at[s],
                device_id=(right,), device_id_type=pl.DeviceIdType.MESH,
            )
            rdma_l = pltpu.make_async_remote_copy(
                src_ref=rsl_ref.at[src], dst_ref=rsl_ref.at[s],
                send_sem=rsl_send.at[s], recv_sem=rsl_recv.at[s],
                device_id=(left,), device_id_type=pl.DeviceIdType.MESH,
            )
            rdma_r.start()
            rdma_l.start()
            return carry

        lax.fori_loop(0, N_HOP, rs_hop, 0)

        p_r = attn_chunk(my, 0)
        p_l = attn_chunk(my, 1)
        last_r = pltpu.make_async_remote_copy(
            src_ref=rsr_ref.at[N_HOP], dst_ref=rsr_ref.at[N_HOP - 1],
            send_sem=rsr_send.at[N_HOP - 1], recv_sem=rsr_recv.at[N_HOP - 1],
            device_id=(right,), device_id_type=pl.DeviceIdType.MESH,
        )
        last_l = pltpu.make_async_remote_copy(
            src_ref=rsl_ref.at[N_HOP], dst_ref=rsl_ref.at[N_HOP - 1],
            send_sem=rsl_send.at[N_HOP - 1], recv_sem=rsl_recv.at[N_HOP - 1],
            device_id=(left,), device_id_type=pl.DeviceIdType.MESH,
        )
        last_r.wait_recv()
        last_l.wait_recv()
        out_ref[0] = rsr_ref[N_HOP - 1].astype(f32) + p_r
        out_ref[1] = rsl_ref[N_HOP - 1].astype(f32) + p_l

        for s in range(N_HOP):
            for ref, ssem, rsem, dev in (
                (rsr_ref, rsr_send, rsr_recv, right),
                (rsl_ref, rsl_send, rsl_recv, left),
            ):
                d = pltpu.make_async_remote_copy(
                    src_ref=ref.at[s], dst_ref=ref.at[s],
                    send_sem=ssem.at[s], recv_sem=rsem.at[s],
                    device_id=(dev,), device_id_type=pl.DeviceIdType.MESH,
                )
                d.wait_send()

    return pl.pallas_call(
        body,
        out_shape=jax.ShapeDtypeStruct((B, S_PER, D), jnp.float32),
        in_specs=[pl.BlockSpec(memory_space=pltpu.VMEM)] * 5,
        out_specs=pl.BlockSpec(memory_space=pltpu.VMEM),
        scratch_shapes=[
            pltpu.VMEM((B, S, D), bf16),
            pltpu.VMEM((B, H_PER, S, DH), bf16),
            pltpu.VMEM((B, H_PER, S, DH), bf16),
            pltpu.VMEM((B, H_PER, S, DH), bf16),
            pltpu.VMEM((N_DEV, S_PER, D), bf16),
            pltpu.VMEM((N_DEV, S_PER, D), bf16),
            pltpu.SemaphoreType.DMA((N_HOP,)),
            pltpu.SemaphoreType.DMA((N_HOP,)),
            pltpu.SemaphoreType.DMA((N_HOP,)),
            pltpu.SemaphoreType.DMA((N_HOP,)),
            pltpu.SemaphoreType.DMA((N_HOP,)),
            pltpu.SemaphoreType.DMA((N_HOP,)),
            pltpu.SemaphoreType.DMA((N_HOP,)),
            pltpu.SemaphoreType.DMA((N_HOP,)),
        ],
        compiler_params=pltpu.CompilerParams(
            collective_id=0, vmem_limit_bytes=110 * 1024 * 1024),
    )(x_b, wq, wk, wv, wo)


# device time: 120440 ns/iter; 2.1181x vs baseline; 1.2264x over previous
import math

import jax
import jax.numpy as jnp
from jax import lax
from jax.experimental import pallas as pl
from jax.experimental.pallas import tpu as pltpu

N_DEV = 8
B = 2
S_PER = 256
S = N_DEV * S_PER
D = 768
H_PER = 4
DH = 64
N_HOP = N_DEV - 1


def kernel(x, Wq, Wk, Wv, Wo):
    bf16 = jnp.bfloat16
    f32 = jnp.float32

    x_b = x.astype(bf16)
    wq = Wq.reshape(D, H_PER, DH).transpose(1, 0, 2).astype(bf16)
    wk = Wk.reshape(D, H_PER, DH).transpose(1, 0, 2).astype(bf16)
    wv = Wv.reshape(D, H_PER, DH).transpose(1, 0, 2).astype(bf16)
    wo = Wo.astype(bf16)

    def body(x_ref, wq_ref, wk_ref, wv_ref, wo_ref, out_ref,
             xg_ref, qb_ref, kb_ref, vb_ref, rsr_ref, rsl_ref,
             agr_send, agr_recv, agl_send, agl_recv,
             rsr_send, rsr_recv, rsl_send, rsl_recv):
        my = lax.axis_index("i")
        left = lax.rem(my + N_DEV - 1, N_DEV)
        right = lax.rem(my + 1, N_DEV)

        barrier = pltpu.get_barrier_semaphore()
        for nbr in (left, right):
            pl.semaphore_signal(barrier, inc=1, device_id=(nbr,),
                                device_id_type=pl.DeviceIdType.MESH)
        pl.semaphore_wait(barrier, 2)

        def project(o, b):
            base = (o * S_PER).astype(f32)
            pos = lax.broadcasted_iota(jnp.int32, (S_PER, DH), 0).astype(f32)
            pos = pos + base
            lanes = lax.broadcasted_iota(jnp.int32, (S_PER, DH), 1)
            even = lanes - lax.rem(lanes, 2)
            theta = pos * jnp.exp(even.astype(f32) * (-math.log(10000.0) / DH))
            cos_t = jnp.cos(theta)
            sin_t = jnp.sin(theta)
            r_row = lax.broadcasted_iota(jnp.int32, (DH, DH), 0)
            r_col = lax.broadcasted_iota(jnp.int32, (DH, DH), 1)
            rot_m = jnp.where(
                (lax.rem(r_col, 2) == 0) & (r_row == r_col + 1), -1.0,
                jnp.where(
                    (lax.rem(r_col, 2) == 1) & (r_row == r_col - 1), 1.0, 0.0),
            ).astype(f32)

            def rope(t):
                return (t * cos_t
                        + jnp.dot(t, rot_m, preferred_element_type=f32) * sin_t)

            xo = xg_ref[b, pl.ds(o * S_PER, S_PER), :]
            for h in range(H_PER):
                q = jnp.dot(xo, wq_ref[h], preferred_element_type=f32)
                k = jnp.dot(xo, wk_ref[h], preferred_element_type=f32)
                v = jnp.dot(xo, wv_ref[h], preferred_element_type=f32)
                qb_ref[b, h, pl.ds(o * S_PER, S_PER), :] = (
                    rope(q) * 0.125).astype(bf16)
                kb_ref[b, h, pl.ds(o * S_PER, S_PER), :] = rope(k).astype(bf16)
                vb_ref[b, h, pl.ds(o * S_PER, S_PER), :] = v.astype(bf16)

        xg_ref[:, pl.ds(my * S_PER, S_PER), :] = x_ref[...]
        project(my, 0)
        project(my, 1)

        def ag_hop(h, carry):
            o_r = lax.rem(my - h + 2 * N_DEV, N_DEV)
            o_l = lax.rem(my + h, N_DEV)
            rdma_r = pltpu.make_async_remote_copy(
                src_ref=xg_ref.at[0, pl.ds(o_r * S_PER, S_PER), :],
                dst_ref=xg_ref.at[0, pl.ds(o_r * S_PER, S_PER), :],
                send_sem=agr_send.at[h],
                recv_sem=agr_recv.at[h],
                device_id=(right,),
                device_id_type=pl.DeviceIdType.MESH,
            )
            rdma_l = pltpu.make_async_remote_copy(
                src_ref=xg_ref.at[1, pl.ds(o_l * S_PER, S_PER), :],
                dst_ref=xg_ref.at[1, pl.ds(o_l * S_PER, S_PER), :],
                send_sem=agl_send.at[h],
                recv_sem=agl_recv.at[h],
                device_id=(left,),
                device_id_type=pl.DeviceIdType.MESH,
            )
            rdma_r.start()
            rdma_l.start()
            @pl.when(h > 0)
            def _():
                project(o_r, 0)
                project(o_l, 1)
            rdma_r.wait()
            rdma_l.wait()
            return carry

        lax.fori_loop(0, N_HOP, ag_hop, 0)
        project(lax.rem(my + 1, N_DEV), 0)
        project(lax.rem(my + N_DEV - 1, N_DEV), 1)

        def attn_chunk(c, b):
            ctxs = []
            for h in range(H_PER):
                qc = qb_ref[b, h, pl.ds(c * S_PER, S_PER), :]
                kh = kb_ref[b, h]
                s_ = lax.dot_general(
                    qc, kh, (((1,), (1,)), ((), ())),
                    preferred_element_type=f32,
                )
                e = jnp.exp(s_)
                den = jnp.sum(e, axis=1, keepdims=True)
                ctx = jnp.dot(e.astype(bf16), vb_ref[b, h],
                              preferred_element_type=f32) / den
                ctxs.append(ctx.astype(bf16))
            ctx = jnp.concatenate(ctxs, axis=1)
            return jnp.dot(ctx, wo_ref[...], preferred_element_type=f32)

        rsr_ref[N_HOP] = jnp.zeros((S_PER, D), bf16)
        rsl_ref[N_HOP] = jnp.zeros((S_PER, D), bf16)

        def rs_hop(s, carry):
            c_r = lax.rem(my - 1 - s + 2 * N_DEV, N_DEV)
            c_l = lax.rem(my + 1 + s, N_DEV)
            sm1 = jnp.maximum(s - 1, 0)
            src = jnp.where(s == 0, N_HOP, sm1)
            p_r = attn_chunk(c_r, 0)
            p_l = attn_chunk(c_l, 1)

            @pl.when(s > 0)
            def _():
                prev_r = pltpu.make_async_remote_copy(
                    src_ref=rsr_ref.at[sm1], dst_ref=rsr_ref.at[sm1],
                    send_sem=rsr_send.at[sm1], recv_sem=rsr_recv.at[sm1],
                    device_id=(right,), device_id_type=pl.DeviceIdType.MESH,
                )
                prev_l = pltpu.make_async_remote_copy(
                    src_ref=rsl_ref.at[sm1], dst_ref=rsl_ref.at[sm1],
                    send_sem=rsl_send.at[sm1], recv_sem=rsl_recv.at[sm1],
                    device_id=(left,), device_id_type=pl.DeviceIdType.MESH,
                )
                prev_r.wait_recv()
                prev_l.wait_recv()

            rsr_ref[src] = (rsr_ref[src].astype(f32) + p_r).astype(bf16)
            rsl_ref[src] = (rsl_ref[src].astype(f32) + p_l).astype(bf16)
            rdma_r = pltpu.make_async_remote_copy(
                src_ref=rsr_ref.at[src], dst_ref=rsr_ref.at[s],
                send_sem=rsr_send.at[s], recv_sem=rsr_recv.at[s],
                device_id=(right,), device_id_type=pl.DeviceIdType.MESH,
            )
            rdma_l = pltpu.make_async_remote_copy(
                src_ref=rsl_ref.at[src], dst_ref=rsl_ref.at[s],
                send_sem=rsl_send.at[s], recv_sem=rsl_recv.at[s],
                device_id=(left,), device_id_type=pl.DeviceIdType.MESH,
            )
            rdma_r.start()
            rdma_l.start()
            return carry

        lax.fori_loop(0, N_HOP, rs_hop, 0)

        p_r = attn_chunk(my, 0)
        p_l = attn_chunk(my, 1)
        last_r = pltpu.make_async_remote_copy(
            src_ref=rsr_ref.at[N_HOP], dst_ref=rsr_ref.at[N_HOP - 1],
            send_sem=rsr_send.at[N_HOP - 1], recv_sem=rsr_recv.at[N_HOP - 1],
            device_id=(right,), device_id_type=pl.DeviceIdType.MESH,
        )
        last_l = pltpu.make_async_remote_copy(
            src_ref=rsl_ref.at[N_HOP], dst_ref=rsl_ref.at[N_HOP - 1],
            send_sem=rsl_send.at[N_HOP - 1], recv_sem=rsl_recv.at[N_HOP - 1],
            device_id=(left,), device_id_type=pl.DeviceIdType.MESH,
        )
        last_r.wait_recv()
        last_l.wait_recv()
        out_ref[0] = rsr_ref[N_HOP - 1].astype(f32) + p_r
        out_ref[1] = rsl_ref[N_HOP - 1].astype(f32) + p_l

        for s in range(N_HOP):
            for ref, ssem, rsem, dev in (
                (rsr_ref, rsr_send, rsr_recv, right),
                (rsl_ref, rsl_send, rsl_recv, left),
            ):
                d = pltpu.make_async_remote_copy(
                    src_ref=ref.at[s], dst_ref=ref.at[s],
                    send_sem=ssem.at[s], recv_sem=rsem.at[s],
                    device_id=(dev,), device_id_type=pl.DeviceIdType.MESH,
                )
                d.wait_send()

    return pl.pallas_call(
        body,
        out_shape=jax.ShapeDtypeStruct((B, S_PER, D), jnp.float32),
        in_specs=[pl.BlockSpec(memory_space=pltpu.VMEM)] * 5,
        out_specs=pl.BlockSpec(memory_space=pltpu.VMEM),
        scratch_shapes=[
            pltpu.VMEM((B, S, D), bf16),
            pltpu.VMEM((B, H_PER, S, DH), bf16),
            pltpu.VMEM((B, H_PER, S, DH), bf16),
            pltpu.VMEM((B, H_PER, S, DH), bf16),
            pltpu.VMEM((N_DEV, S_PER, D), bf16),
            pltpu.VMEM((N_DEV, S_PER, D), bf16),
            pltpu.SemaphoreType.DMA((N_HOP,)),
            pltpu.SemaphoreType.DMA((N_HOP,)),
            pltpu.SemaphoreType.DMA((N_HOP,)),
            pltpu.SemaphoreType.DMA((N_HOP,)),
            pltpu.SemaphoreType.DMA((N_HOP,)),
            pltpu.SemaphoreType.DMA((N_HOP,)),
            pltpu.SemaphoreType.DMA((N_HOP,)),
            pltpu.SemaphoreType.DMA((N_HOP,)),
        ],
        compiler_params=pltpu.CompilerParams(
            collective_id=0, vmem_limit_bytes=110 * 1024 * 1024),
    )(x_b, wq, wk, wv, wo)


# device time: 107061 ns/iter; 2.3828x vs baseline; 1.1250x over previous
import math

import jax
import jax.numpy as jnp
from jax import lax
from jax.experimental import pallas as pl
from jax.experimental.pallas import tpu as pltpu

N_DEV = 8
B = 2
S_PER = 256
S = N_DEV * S_PER
D = 768
H_PER = 4
DH = 64
N_HOP = N_DEV - 1
N_SUB = 2
D_SUB = D // N_SUB


def kernel(x, Wq, Wk, Wv, Wo):
    bf16 = jnp.bfloat16
    f32 = jnp.float32

    x_b = x.astype(bf16)
    wq = Wq.reshape(D, H_PER, DH).transpose(1, 0, 2).astype(bf16)
    wk = Wk.reshape(D, H_PER, DH).transpose(1, 0, 2).astype(bf16)
    wv = Wv.reshape(D, H_PER, DH).transpose(1, 0, 2).astype(bf16)
    wo = Wo.astype(bf16)

    def body(x_ref, wq_ref, wk_ref, wv_ref, wo_ref, out_ref,
             xg_ref, qb_ref, kb_ref, vb_ref, rsr_ref, rsl_ref,
             agr_send, agr_recv, agl_send, agl_recv,
             rsr_send, rsr_recv, rsl_send, rsl_recv):
        my = lax.axis_index("i")
        left = lax.rem(my + N_DEV - 1, N_DEV)
        right = lax.rem(my + 1, N_DEV)

        barrier = pltpu.get_barrier_semaphore()
        for nbr in (left, right):
            pl.semaphore_signal(barrier, inc=1, device_id=(nbr,),
                                device_id_type=pl.DeviceIdType.MESH)
        pl.semaphore_wait(barrier, 2)

        def ag_rdma(h, b, o, sub):
            dev = right if b == 0 else left
            send = (agr_send if b == 0 else agl_send).at[h, sub]
            recv = (agr_recv if b == 0 else agl_recv).at[h, sub]
            sl = xg_ref.at[b, pl.ds(o * S_PER, S_PER),
                           pl.ds(sub * D_SUB, D_SUB)]
            return pltpu.make_async_remote_copy(
                src_ref=sl, dst_ref=sl, send_sem=send, recv_sem=recv,
                device_id=(dev,), device_id_type=pl.DeviceIdType.MESH,
            )

        def project(o, b):
            base = (o * S_PER).astype(f32)
            pos = lax.broadcasted_iota(jnp.int32, (S_PER, DH), 0).astype(f32)
            pos = pos + base
            lanes = lax.broadcasted_iota(jnp.int32, (S_PER, DH), 1)
            even = lanes - lax.rem(lanes, 2)
            theta = pos * jnp.exp(even.astype(f32) * (-math.log(10000.0) / DH))
            cos_t = jnp.cos(theta)
            sin_t = jnp.sin(theta)
            r_row = lax.broadcasted_iota(jnp.int32, (DH, DH), 0)
            r_col = lax.broadcasted_iota(jnp.int32, (DH, DH), 1)
            rot_m = jnp.where(
                (lax.rem(r_col, 2) == 0) & (r_row == r_col + 1), -1.0,
                jnp.where(
                    (lax.rem(r_col, 2) == 1) & (r_row == r_col - 1), 1.0, 0.0),
            ).astype(f32)

            def rope(t):
                return (t * cos_t
                        + jnp.dot(t, rot_m, preferred_element_type=f32) * sin_t)

            xo = xg_ref[b, pl.ds(o * S_PER, S_PER), :]
            for h in range(H_PER):
                q = jnp.dot(xo, wq_ref[h], preferred_element_type=f32)
                k = jnp.dot(xo, wk_ref[h], preferred_element_type=f32)
                v = jnp.dot(xo, wv_ref[h], preferred_element_type=f32)
                qb_ref[b, h, pl.ds(o * S_PER, S_PER), :] = (
                    rope(q) * (0.125 * math.log2(math.e))).astype(bf16)
                kb_ref[b, h, pl.ds(o * S_PER, S_PER), :] = rope(k).astype(bf16)
                vb_ref[b, h, pl.ds(o * S_PER, S_PER), :] = jnp.concatenate(
                    [v, jnp.ones((S_PER, 1), f32),
                     jnp.zeros((S_PER, DH - 1), f32)], axis=1).astype(bf16)

        xg_ref[:, pl.ds(my * S_PER, S_PER), :] = x_ref[...]

        def ag_hop(h, carry):
            o_r = lax.rem(my - h + 2 * N_DEV, N_DEV)
            o_l = lax.rem(my + h, N_DEV)
            hm1 = jnp.maximum(h - 1, 0)
            o_r_prev = lax.rem(o_r + 1, N_DEV)
            o_l_prev = lax.rem(o_l + N_DEV - 1, N_DEV)
            for sub in range(N_SUB):
                @pl.when(h > 0)
                def _():
                    ag_rdma(hm1, 0, o_r_prev, sub).wait()
                ag_rdma(h, 0, o_r, sub).start()

                @pl.when(h > 0)
                def _():
                    ag_rdma(hm1, 1, o_l_prev, sub).wait()
                ag_rdma(h, 1, o_l, sub).start()
            project(o_r, 0)
            project(o_l, 1)
            return carry

        lax.fori_loop(0, N_HOP, ag_hop, 0)
        o_r_last = lax.rem(my + 1, N_DEV)
        o_l_last = lax.rem(my + N_DEV - 1, N_DEV)
        for sub in range(N_SUB):
            ag_rdma(N_HOP - 1, 0, lax.rem(o_r_last + 1, N_DEV), sub).wait()
            ag_rdma(N_HOP - 1, 1, lax.rem(o_l_last + N_DEV - 1, N_DEV),
                    sub).wait()
        project(o_r_last, 0)
        project(o_l_last, 1)

        def attn_chunk(c, b):
            ctxs = []
            for h in range(H_PER):
                qc = qb_ref[b, h, pl.ds(c * S_PER, S_PER), :]
                kh = kb_ref[b, h]
                s_ = lax.dot_general(
                    qc, kh, (((1,), (1,)), ((), ())),
                    preferred_element_type=f32,
                )
                e = jnp.exp2(s_)
                o_ = jnp.dot(e.astype(bf16), vb_ref[b, h],
                             preferred_element_type=f32)
                ctx = o_[:, :DH] / o_[:, DH:DH + 1]
                ctxs.append(ctx.astype(bf16))
            ctx = jnp.concatenate(ctxs, axis=1)
            return jnp.dot(ctx, wo_ref[...], preferred_element_type=f32)

        def rs_rdma(s, b, src, dst, sub):
            dev = right if b == 0 else left
            ref = rsr_ref if b == 0 else rsl_ref
            send = (rsr_send if b == 0 else rsl_send).at[s, sub]
            recv = (rsr_recv if b == 0 else rsl_recv).at[s, sub]
            return pltpu.make_async_remote_copy(
                src_ref=ref.at[src, :, pl.ds(sub * D_SUB, D_SUB)],
                dst_ref=ref.at[dst, :, pl.ds(sub * D_SUB, D_SUB)],
                send_sem=send, recv_sem=recv,
                device_id=(dev,), device_id_type=pl.DeviceIdType.MESH,
            )

        rsr_ref[N_HOP] = jnp.zeros((S_PER, D), bf16)
        rsl_ref[N_HOP] = jnp.zeros((S_PER, D), bf16)

        def rs_hop(s, carry):
            c_r = lax.rem(my - 1 - s + 2 * N_DEV, N_DEV)
            c_l = lax.rem(my + 1 + s, N_DEV)
            sm1 = jnp.maximum(s - 1, 0)
            src = jnp.where(s == 0, N_HOP, sm1)
            p_r = attn_chunk(c_r, 0)
            p_l = attn_chunk(c_l, 1)
            for sub in range(N_SUB):
                dsl = pl.ds(sub * D_SUB, D_SUB)

                lo, hi = sub * D_SUB, (sub + 1) * D_SUB

                @pl.when(s > 0)
                def _():
                    rs_rdma(sm1, 0, sm1, sm1, sub).wait_recv()
                rsr_ref[src, :, dsl] = (
                    rsr_ref[src, :, dsl].astype(f32) + p_r[:, lo:hi]
                ).astype(bf16)
                rs_rdma(s, 0, src, s, sub).start()

                @pl.when(s > 0)
                def _():
                    rs_rdma(sm1, 1, sm1, sm1, sub).wait_recv()
                rsl_ref[src, :, dsl] = (
                    rsl_ref[src, :, dsl].astype(f32) + p_l[:, lo:hi]
                ).astype(bf16)
                rs_rdma(s, 1, src, s, sub).start()
            return carry

        lax.fori_loop(0, N_HOP, rs_hop, 0)

        p_r = attn_chunk(my, 0)
        p_l = attn_chunk(my, 1)
        for sub in range(N_SUB):
            rs_rdma(N_HOP - 1, 0, N_HOP - 1, N_HOP - 1, sub).wait_recv()
            rs_rdma(N_HOP - 1, 1, N_HOP - 1, N_HOP - 1, sub).wait_recv()
        out_ref[0] = rsr_ref[N_HOP - 1].astype(f32) + p_r
        out_ref[1] = rsl_ref[N_HOP - 1].astype(f32) + p_l

        for s in range(N_HOP):
            for sub in range(N_SUB):
                rs_rdma(s, 0, s, s, sub).wait_send()
                rs_rdma(s, 1, s, s, sub).wait_send()

    return pl.pallas_call(
        body,
        out_shape=jax.ShapeDtypeStruct((B, S_PER, D), jnp.float32),
        in_specs=[pl.BlockSpec(memory_space=pltpu.VMEM)] * 5,
        out_specs=pl.BlockSpec(memory_space=pltpu.VMEM),
        scratch_shapes=[
            pltpu.VMEM((B, S, D), bf16),
            pltpu.VMEM((B, H_PER, S, DH), bf16),
            pltpu.VMEM((B, H_PER, S, DH), bf16),
            pltpu.VMEM((B, H_PER, S, 2 * DH), bf16),
            pltpu.VMEM((N_DEV, S_PER, D), bf16),
            pltpu.VMEM((N_DEV, S_PER, D), bf16),
            pltpu.SemaphoreType.DMA((N_HOP, N_SUB)),
            pltpu.SemaphoreType.DMA((N_HOP, N_SUB)),
            pltpu.SemaphoreType.DMA((N_HOP, N_SUB)),
            pltpu.SemaphoreType.DMA((N_HOP, N_SUB)),
            pltpu.SemaphoreType.DMA((N_HOP, N_SUB)),
            pltpu.SemaphoreType.DMA((N_HOP, N_SUB)),
            pltpu.SemaphoreType.DMA((N_HOP, N_SUB)),
            pltpu.SemaphoreType.DMA((N_HOP, N_SUB)),
        ],
        compiler_params=pltpu.CompilerParams(
            collective_id=0, vmem_limit_bytes=110 * 1024 * 1024),
    )(x_b, wq, wk, wv, wo)
